# baseline (speedup 1.0000x reference)
import jax
import jax.numpy as jnp
from jax import lax
from jax.experimental import pallas as pl
from jax.experimental.pallas import tpu as pltpu
from jax.experimental.pallas import tpu_sc as plsc

NC = 2
NS = 16
NW = NC * NS
LANES = 16
C = 128
BC = 8


def _sc_segment_sum(n_feat, sdw_r, n_chunks):
    N, D = n_feat.shape
    rpt = -(-N // (NS * 8)) * 8
    Npad = rpt * NS
    n_full = rpt // C
    tail = rpt - n_full * C
    mesh = plsc.VectorSubcoreMesh(
        core_axis_name="c", subcore_axis_name="s",
        num_cores=NC, num_subcores=NS)

    def body(nfeat_hbm, sdw_hbm, out_hbm,
             sdw_v, rows_0, rows_1, neigh_sh, gsem, ssem):
        cid = lax.axis_index("c")
        sid = lax.axis_index("s")
        wid = sid * NC + cid
        bufs = (rows_0, rows_1)

        zeros = jnp.zeros((LANES,), jnp.float32)

        def zrow(i, _):
            for k in range(D // LANES):
                rows_0[i, pl.ds(k * LANES, LANES)] = zeros
            return 0

        lax.fori_loop(0, C, zrow, 0)
        base = sid * rpt
        for k in range(n_full):
            pltpu.sync_copy(rows_0, neigh_sh.at[pl.ds(base + k * C, C)])
        if tail:
            pltpu.sync_copy(rows_0.at[pl.ds(0, tail)],
                            neigh_sh.at[pl.ds(base + n_full * C, tail)])
        plsc.subcore_barrier()

        def scale(buf, c):
            def group(g, _):
                wv = plsc.bitcast(
                    sdw_v[3 * c + 2, pl.ds(g * LANES, LANES)], jnp.float32)
                for l in range(LANES):
                    ws = wv[l]
                    i = g * LANES + l
                    for k in range(D // LANES):
                        sl = pl.ds(k * LANES, LANES)
                        buf[i, sl] = buf[i, sl] * ws
                return 0

            lax.fori_loop(0, C // LANES, group, 0)

        def block(bk, _):
            pltpu.sync_copy(
                sdw_hbm.at[wid, pl.ds(bk * 3 * BC, 3 * BC)], sdw_v)
            pltpu.async_copy(nfeat_hbm.at[sdw_v.at[0]], rows_0, gsem)
            for c in range(BC):
                cur = bufs[c % 2]
                pltpu.make_async_copy(
                    nfeat_hbm.at[sdw_v.at[3 * c]], cur, gsem).wait()
                if c >= 1:
                    prev = bufs[(c - 1) % 2]
                    pltpu.make_async_copy(
                        prev, neigh_sh.at[sdw_v.at[3 * (c - 1) + 1]],
                        ssem).wait()
                if c + 1 < BC:
                    pltpu.async_copy(
                        nfeat_hbm.at[sdw_v.at[3 * (c + 1)]],
                        bufs[(c + 1) % 2], gsem)
                scale(cur, c)
                pltpu.async_copy(
                    cur, neigh_sh.at[sdw_v.at[3 * c + 1]], ssem, add=True)
            pltpu.make_async_copy(
                bufs[(BC - 1) % 2],
                neigh_sh.at[sdw_v.at[3 * (BC - 1) + 1]], ssem).wait()
            return 0

        lax.fori_loop(0, n_chunks // BC, block, 0)

        plsc.subcore_barrier()
        pltpu.sync_copy(neigh_sh.at[pl.ds(base, rpt)],
                        out_hbm.at[cid, pl.ds(base, rpt)])

    run = pl.kernel(
        body,
        out_type=jax.ShapeDtypeStruct((NC, Npad, D), jnp.float32),
        mesh=mesh,
        compiler_params=pltpu.CompilerParams(needs_layout_passes=False),
        scratch_types=[
            pltpu.VMEM((3 * BC, C), jnp.int32),
            pltpu.VMEM((C, D), jnp.float32),
            pltpu.VMEM((C, D), jnp.float32),
            pltpu.VMEM_SHARED((Npad, D), jnp.float32),
            pltpu.SemaphoreType.DMA,
            pltpu.SemaphoreType.DMA,
        ],
    )
    return run(n_feat, sdw_r)


def _tc_mlp(n_feat, partials, W1, b1, W2, b2):
    N, D = n_feat.shape
    BLK = 2000
    grid = N // BLK

    def body(nf_ref, pp_ref, w1_ref, b1_ref, w2_ref, b2_ref, out_ref):
        rst = nf_ref[...] + pp_ref[0] + pp_ref[1]
        h = jnp.dot(rst, w1_ref[...], preferred_element_type=jnp.float32)
        h = jnp.maximum(h + b1_ref[...], 0.0)
        o = jnp.dot(h, w2_ref[...], preferred_element_type=jnp.float32)
        out_ref[...] = o + b2_ref[...]

    return pl.pallas_call(
        body,
        grid=(grid,),
        in_specs=[
            pl.BlockSpec((BLK, D), lambda i: (i, 0)),
            pl.BlockSpec((NC, BLK, D), lambda i: (0, i, 0)),
            pl.BlockSpec((D, D), lambda i: (0, 0)),
            pl.BlockSpec((1, D), lambda i: (0, 0)),
            pl.BlockSpec((D, D), lambda i: (0, 0)),
            pl.BlockSpec((1, D), lambda i: (0, 0)),
        ],
        out_specs=pl.BlockSpec((BLK, D), lambda i: (i, 0)),
        out_shape=jax.ShapeDtypeStruct((N, D), jnp.float32),
    )(n_feat, partials, W1, b1.reshape(1, D), W2, b2.reshape(1, D))


@jax.jit
def kernel(n_feat, e_feat, edge_weight, edge_index, W1, b1, W2, b2):
    del e_feat
    N, D = n_feat.shape
    E = edge_index.shape[1]
    epw = -(-E // NW)
    epw = -(-epw // (BC * C)) * (BC * C)
    E_pad = epw * NW
    pad = E_pad - E

    sd = edge_index.astype(jnp.int32)
    w = edge_weight[:, 0].astype(jnp.float32)
    if pad:
        sd = jnp.concatenate([sd, jnp.zeros((2, pad), jnp.int32)], axis=1)
        w = jnp.concatenate([w, jnp.zeros((pad,), jnp.float32)])

    n_chunks = epw // C
    src_r = sd[0].reshape(NW, n_chunks, C)
    dst_r = sd[1].reshape(NW, n_chunks, C)
    w_r = lax.bitcast_convert_type(
        w.reshape(NW, n_chunks, C), jnp.int32)
    sdw_r = jnp.stack([src_r, dst_r, w_r], axis=2)
    sdw_r = sdw_r.reshape(NW, 3 * n_chunks, C)

    partials = _sc_segment_sum(n_feat, sdw_r, n_chunks)
    return _tc_mlp(n_feat, partials, W1, b1, W2, b2)


# speedup vs baseline: 1.3060x; 1.3060x over previous
import jax
import jax.numpy as jnp
from jax import lax
from jax.experimental import pallas as pl
from jax.experimental.pallas import tpu as pltpu
from jax.experimental.pallas import tpu_sc as plsc

NC = 2
NS = 16
NW = NC * NS
LANES = 16
C = 128


def _sc_segment_sum(n_feat, sd_r, w_r, n_chunks):
    N, D = n_feat.shape
    rpt = -(-N // (NS * 8)) * 8
    Npad = rpt * NS
    n_full = rpt // C
    tail = rpt - n_full * C
    mesh = plsc.VectorSubcoreMesh(
        core_axis_name="c", subcore_axis_name="s",
        num_cores=NC, num_subcores=NS)

    def body(nfeat_hbm, sd_hbm, w_hbm, out_hbm,
             src_v, dst_v, w_v, rows_v, neigh_sh, sem):
        cid = lax.axis_index("c")
        sid = lax.axis_index("s")
        wid = sid * NC + cid

        pltpu.sync_copy(sd_hbm.at[0, wid], src_v)
        pltpu.sync_copy(sd_hbm.at[1, wid], dst_v)
        pltpu.sync_copy(w_hbm.at[wid], w_v)

        zeros = jnp.zeros((LANES,), jnp.float32)

        def zrow(i, _):
            for k in range(D // LANES):
                rows_v[i, pl.ds(k * LANES, LANES)] = zeros
            return 0

        lax.fori_loop(0, C, zrow, 0)
        base = sid * rpt
        for k in range(n_full):
            pltpu.sync_copy(rows_v, neigh_sh.at[pl.ds(base + k * C, C)])
        if tail:
            pltpu.sync_copy(rows_v.at[pl.ds(0, tail)],
                            neigh_sh.at[pl.ds(base + n_full * C, tail)])
        plsc.subcore_barrier()

        def step(j, _):
            pltpu.async_copy(nfeat_hbm.at[src_v.at[j]], rows_v, sem).wait()

            def group(g, _):
                wv = w_v[j, pl.ds(g * LANES, LANES)]
                for l in range(LANES):
                    ws = wv[l]
                    i = g * LANES + l
                    for k in range(D // LANES):
                        sl = pl.ds(k * LANES, LANES)
                        rows_v[i, sl] = rows_v[i, sl] * ws
                return 0

            lax.fori_loop(0, C // LANES, group, 0)
            pltpu.sync_copy(rows_v, neigh_sh.at[dst_v.at[j]], add=True)
            return 0

        lax.fori_loop(0, n_chunks, step, 0)

        plsc.subcore_barrier()
        pltpu.sync_copy(neigh_sh.at[pl.ds(base, rpt)],
                        out_hbm.at[cid, pl.ds(base, rpt)])

    run = pl.kernel(
        body,
        out_type=jax.ShapeDtypeStruct((NC, Npad, D), jnp.float32),
        mesh=mesh,
        scratch_types=[
            pltpu.VMEM((n_chunks, C), jnp.int32),
            pltpu.VMEM((n_chunks, C), jnp.int32),
            pltpu.VMEM((n_chunks, C), jnp.float32),
            pltpu.VMEM((C, D), jnp.float32),
            pltpu.VMEM_SHARED((Npad, D), jnp.float32),
            pltpu.SemaphoreType.DMA,
        ],
    )
    return run(n_feat, sd_r, w_r)


def _tc_mlp(n_feat, partials, W1, b1, W2, b2):
    N, D = n_feat.shape
    BLK = 2000
    grid = N // BLK

    def body(nf_ref, pp_ref, w1_ref, b1_ref, w2_ref, b2_ref, out_ref):
        rst = nf_ref[...] + pp_ref[0] + pp_ref[1]
        h = jnp.dot(rst, w1_ref[...], preferred_element_type=jnp.float32)
        h = jnp.maximum(h + b1_ref[...], 0.0)
        o = jnp.dot(h, w2_ref[...], preferred_element_type=jnp.float32)
        out_ref[...] = o + b2_ref[...]

    return pl.pallas_call(
        body,
        grid=(grid,),
        in_specs=[
            pl.BlockSpec((BLK, D), lambda i: (i, 0)),
            pl.BlockSpec((NC, BLK, D), lambda i: (0, i, 0)),
            pl.BlockSpec((D, D), lambda i: (0, 0)),
            pl.BlockSpec((1, D), lambda i: (0, 0)),
            pl.BlockSpec((D, D), lambda i: (0, 0)),
            pl.BlockSpec((1, D), lambda i: (0, 0)),
        ],
        out_specs=pl.BlockSpec((BLK, D), lambda i: (i, 0)),
        out_shape=jax.ShapeDtypeStruct((N, D), jnp.float32),
    )(n_feat, partials, W1, b1.reshape(1, D), W2, b2.reshape(1, D))


@jax.jit
def kernel(n_feat, e_feat, edge_weight, edge_index, W1, b1, W2, b2):
    del e_feat
    N, D = n_feat.shape
    E = edge_index.shape[1]
    epw = -(-E // NW)
    epw = -(-epw // C) * C
    E_pad = epw * NW
    pad = E_pad - E

    sd = edge_index.astype(jnp.int32)
    w = edge_weight[:, 0].astype(jnp.float32)
    if pad:
        sd = jnp.concatenate([sd, jnp.zeros((2, pad), jnp.int32)], axis=1)
        w = jnp.concatenate([w, jnp.zeros((pad,), jnp.float32)])

    n_chunks = epw // C
    sd_r = sd.reshape(2, NW, n_chunks, C)
    w_r = w.reshape(NW, n_chunks, C)

    partials = _sc_segment_sum(n_feat, sd_r, w_r, n_chunks)
    return _tc_mlp(n_feat, partials, W1, b1, W2, b2)
